# Initial kernel scaffold; baseline (speedup 1.0000x reference)
import jax
import jax.numpy as jnp
from jax.experimental import pallas as pl


def kernel(x, edge_index, W1, b1, W2, b2, Wn, bn, We, be):
    raise NotImplementedError("write your pallas kernel here")


# speedup vs baseline: 17.0133x; 17.0133x over previous
import functools

import jax
import jax.numpy as jnp
from jax import lax
from jax.experimental import pallas as pl
from jax.experimental.pallas import tpu as pltpu
from jax.experimental.pallas import tpu_sc as plsc

N = 10000
E = 320000
D = 128
NC = 2
NS = 16
NW = NC * NS
EPW = E // NW
CHUNK = 128
NFULL = EPW // CHUNK
TAIL = EPW - NFULL * CHUNK
DEG_PAD = NS * 640
NPAD = NS * 640

_MESH = plsc.VectorSubcoreMesh(core_axis_name="c", subcore_axis_name="s")
_F32 = jnp.float32
_I32 = jnp.int32


@functools.partial(
    pl.kernel,
    out_type=jax.ShapeDtypeStruct((NC * DEG_PAD,), _F32),
    mesh=_MESH,
    compiler_params=pltpu.CompilerParams(
        needs_layout_passes=False, use_tc_tiling_on_sc=False),
    scratch_types=[
        pltpu.VMEM((CHUNK,), _I32),
        pltpu.VMEM((CHUNK,), _F32),
        pltpu.VMEM((TAIL,), _I32),
        pltpu.VMEM((TAIL,), _F32),
        pltpu.VMEM((640,), _F32),
        pltpu.VMEM_SHARED((DEG_PAD,), _F32),
    ],
)
def _sc_degree(dst_hbm, out_hbm, dstv, onesv, dstt, onest, zbuf, acc):
    c = lax.axis_index("c")
    s = lax.axis_index("s")
    base = (c * NS + s) * EPW

    zero16 = jnp.zeros((16,), _F32)
    one16 = jnp.ones((16,), _F32)
    for j in range(640 // 16):
        zbuf[pl.ds(j * 16, 16)] = zero16
    for j in range(CHUNK // 16):
        onesv[pl.ds(j * 16, 16)] = one16
    onest[pl.ds(0, 16)] = one16
    pltpu.sync_copy(zbuf, acc.at[pl.ds(s * 640, 640)])
    plsc.subcore_barrier()

    def body(i, _):
        pltpu.sync_copy(dst_hbm.at[pl.ds(base + i * CHUNK, CHUNK)], dstv)
        pltpu.sync_copy(onesv, acc.at[dstv], add=True)
        return ()

    lax.fori_loop(0, NFULL, body, ())
    pltpu.sync_copy(dst_hbm.at[pl.ds(base + NFULL * CHUNK, TAIL)], dstt)
    pltpu.sync_copy(onest, acc.at[dstt], add=True)

    plsc.subcore_barrier()
    pltpu.sync_copy(acc.at[pl.ds(s * 640, 640)],
                    out_hbm.at[pl.ds(c * DEG_PAD + s * 640, 640)])


@functools.partial(
    pl.kernel,
    out_type=jax.ShapeDtypeStruct((NC, NPAD, D), _F32),
    mesh=_MESH,
    compiler_params=pltpu.CompilerParams(
        needs_layout_passes=False, use_tc_tiling_on_sc=False),
    scratch_types=[
        pltpu.VMEM((CHUNK,), _I32),
        pltpu.VMEM((CHUNK,), _I32),
        pltpu.VMEM((CHUNK,), _I32),
        pltpu.VMEM((CHUNK,), _I32),
        pltpu.VMEM((CHUNK, D), _F32),
        pltpu.VMEM((CHUNK, D), _F32),
        pltpu.VMEM((TAIL,), _I32),
        pltpu.VMEM((TAIL,), _I32),
        pltpu.VMEM_SHARED((NPAD, D), _F32),
        pltpu.SemaphoreType.DMA,
        pltpu.SemaphoreType.DMA,
        pltpu.SemaphoreType.DMA,
        pltpu.SemaphoreType.DMA,
        pltpu.SemaphoreType.DMA,
        pltpu.SemaphoreType.DMA,
    ],
)
def _sc_aggregate(table_hbm, src_hbm, dst_hbm, out_hbm,
                  src0, src1, dst0, dst1, rows0, rows1, srct, dstt, acc,
                  ss0, ss1, sd0, sd1, sg0, sg1):
    c = lax.axis_index("c")
    s = lax.axis_index("s")
    base = (c * NS + s) * EPW
    srcbufs = (src0, src1)
    dstbufs = (dst0, dst1)
    rowbufs = (rows0, rows1)
    sss = (ss0, ss1)
    sds = (sd0, sd1)
    sgs = (sg0, sg1)

    def idx_copies(i, b):
        return (
            pltpu.make_async_copy(
                src_hbm.at[pl.ds(base + i * CHUNK, CHUNK)], srcbufs[b], sss[b]),
            pltpu.make_async_copy(
                dst_hbm.at[pl.ds(base + i * CHUNK, CHUNK)], dstbufs[b], sds[b]),
        )

    def gather_copy(b):
        return pltpu.make_async_copy(
            table_hbm.at[srcbufs[b]], rowbufs[b], sgs[b])

    zero16 = jnp.zeros((16,), _F32)

    def zbody(i, _):
        for j in range(D // 16):
            rows0[i, pl.ds(j * 16, 16)] = zero16
        return ()

    lax.fori_loop(0, CHUNK, zbody, ())
    for k in range(5):
        pltpu.sync_copy(rows0,
                        acc.at[pl.ds(s * 640 + k * CHUNK, CHUNK), :])

    for cp in idx_copies(0, 0):
        cp.start()
    for cp in idx_copies(1, 1):
        cp.start()
    for cp in idx_copies(0, 0):
        cp.wait()

    plsc.subcore_barrier()
    gather_copy(0).start()

    def body(i, _):
        b = lax.rem(i, 2)
        for bb in range(2):
            @pl.when(b == bb)
            def _():
                @pl.when(i + 1 < NFULL)
                def _():
                    for cp in idx_copies(i + 1, 1 - bb):
                        cp.wait()
                    gather_copy(1 - bb).start()
                gather_copy(bb).wait()
                pltpu.sync_copy(rowbufs[bb], acc.at[dstbufs[bb]], add=True)

                @pl.when(i + 2 < NFULL)
                def _():
                    for cp in idx_copies(i + 2, bb):
                        cp.start()
        return ()

    lax.fori_loop(0, NFULL, body, ())

    pltpu.sync_copy(src_hbm.at[pl.ds(base + NFULL * CHUNK, TAIL)], srct)
    pltpu.sync_copy(dst_hbm.at[pl.ds(base + NFULL * CHUNK, TAIL)], dstt)
    pltpu.async_copy(
        table_hbm.at[srct], rows0.at[pl.ds(0, TAIL), :], sg0).wait()
    pltpu.sync_copy(rows0.at[pl.ds(0, TAIL), :], acc.at[dstt], add=True)

    plsc.subcore_barrier()
    pltpu.sync_copy(acc.at[pl.ds(s * 640, 640), :],
                    out_hbm.at[c, pl.ds(s * 640, 640), :])


@functools.partial(
    pl.kernel,
    out_type=jax.ShapeDtypeStruct((2 * E,), _F32),
    mesh=_MESH,
    compiler_params=pltpu.CompilerParams(
        needs_layout_passes=False, use_tc_tiling_on_sc=False),
    scratch_types=[
        pltpu.VMEM((2 * N,), _F32),
        pltpu.VMEM((2 * N,), _F32),
        pltpu.VMEM((EPW,), _I32),
        pltpu.VMEM((EPW,), _I32),
        pltpu.VMEM((2 * EPW,), _F32),
    ],
)
def _sc_edge_head(es_hbm, ed_hbm, src_hbm, dst_hbm, out_hbm,
                  esv, edv, srcv, dstv, outv):
    c = lax.axis_index("c")
    s = lax.axis_index("s")
    base = (c * NS + s) * EPW
    pltpu.sync_copy(es_hbm, esv)
    pltpu.sync_copy(ed_hbm, edv)
    pltpu.sync_copy(src_hbm.at[pl.ds(base, EPW)], srcv)
    pltpu.sync_copy(dst_hbm.at[pl.ds(base, EPW)], dstv)

    lane = lax.iota(_I32, 16)
    ones16 = jnp.ones((16,), _I32)

    def body(j, _):
        off = pl.multiple_of(j * 16, 8)
        s2 = srcv[pl.ds(off, 16)] * 2
        d2 = dstv[pl.ds(off, 16)] * 2
        o0 = plsc.load_gather(esv, [s2]) + plsc.load_gather(edv, [d2])
        o1 = (plsc.load_gather(esv, [s2 + ones16]) +
              plsc.load_gather(edv, [d2 + ones16]))
        e2 = (lane + off) * 2
        plsc.store_scatter(outv, [e2], o0)
        plsc.store_scatter(outv, [e2 + ones16], o1)
        return ()

    lax.fori_loop(0, EPW // 16, body, ())
    pltpu.sync_copy(outv, out_hbm.at[pl.ds(2 * base, 2 * EPW)])


_RB = 1000
_GRID = N // _RB


def _row_spec(width):
    return pl.BlockSpec((_RB, width), lambda i: (i, 0))


def _full_spec(r, cdim):
    return pl.BlockSpec((r, cdim), lambda i: (0, 0))


def _dinv(p0, p1):
    return lax.rsqrt(p0 + p1 + 1.0)


def _tc_scale1_body(x_ref, w_ref, p0_ref, p1_ref, out_ref):
    dinv = _dinv(p0_ref[...], p1_ref[...])
    xw = jnp.dot(x_ref[...], w_ref[...], preferred_element_type=_F32)
    out_ref[...] = xw * dinv


def _tc_mid_body(a0_ref, a1_ref, s1_ref, p0_ref, p1_ref, w_ref, b1_ref,
                 out_ref):
    dinv = _dinv(p0_ref[...], p1_ref[...])
    pre = dinv * (a0_ref[...] + a1_ref[...] + s1_ref[...]) + b1_ref[...]
    h = jnp.maximum(pre, 0.0)
    out_ref[...] = dinv * jnp.dot(h, w_ref[...], preferred_element_type=_F32)


def _tc_head_body(a0_ref, a1_ref, s2_ref, p0_ref, p1_ref, b2_ref,
                  wn_ref, bn_ref, ws_ref, wd_ref, be_ref,
                  nx_ref, es_ref, ed_ref):
    dinv = _dinv(p0_ref[...], p1_ref[...])
    emb = dinv * (a0_ref[...] + a1_ref[...] + s2_ref[...]) + b2_ref[...]
    nx_ref[...] = jnp.dot(emb, wn_ref[...], preferred_element_type=_F32) + bn_ref[...]
    es_ref[...] = jnp.dot(emb, ws_ref[...], preferred_element_type=_F32) + be_ref[...]
    ed_ref[...] = jnp.dot(emb, wd_ref[...], preferred_element_type=_F32)


def kernel(x, edge_index, W1, b1, W2, b2, Wn, bn, We, be):
    src = edge_index[0].astype(_I32)
    dst = edge_index[1].astype(_I32)

    deg_parts = _sc_degree(dst)
    p0 = deg_parts[:N].reshape(N, 1)
    p1 = deg_parts[DEG_PAD:DEG_PAD + N].reshape(N, 1)

    scaled1 = pl.pallas_call(
        _tc_scale1_body,
        grid=(_GRID,),
        in_specs=[_row_spec(D), _full_spec(D, D), _row_spec(1), _row_spec(1)],
        out_specs=_row_spec(D),
        out_shape=jax.ShapeDtypeStruct((N, D), _F32),
    )(x, W1, p0, p1)

    agg1 = _sc_aggregate(scaled1, src, dst)

    scaled2 = pl.pallas_call(
        _tc_mid_body,
        grid=(_GRID,),
        in_specs=[_row_spec(D), _row_spec(D), _row_spec(D), _row_spec(1),
                  _row_spec(1), _full_spec(D, D), _full_spec(1, D)],
        out_specs=_row_spec(D),
        out_shape=jax.ShapeDtypeStruct((N, D), _F32),
    )(agg1[0], agg1[1], scaled1, p0, p1, W2, b1.reshape(1, D))

    agg2 = _sc_aggregate(scaled2, src, dst)

    node_x, es, ed = pl.pallas_call(
        _tc_head_body,
        grid=(_GRID,),
        in_specs=[_row_spec(D), _row_spec(D), _row_spec(D), _row_spec(1),
                  _row_spec(1), _full_spec(1, D), _full_spec(D, 2),
                  _full_spec(1, 2), _full_spec(D, 2), _full_spec(D, 2),
                  _full_spec(1, 2)],
        out_specs=[_row_spec(2), _row_spec(2), _row_spec(2)],
        out_shape=[jax.ShapeDtypeStruct((N, 2), _F32),
                   jax.ShapeDtypeStruct((N, 2), _F32),
                   jax.ShapeDtypeStruct((N, 2), _F32)],
    )(agg2[0], agg2[1], scaled2, p0, p1, b2.reshape(1, D),
      Wn, bn.reshape(1, 2), We[:D], We[D:], be.reshape(1, 2))

    edge_flat = _sc_edge_head(es.reshape(-1), ed.reshape(-1), src, dst)
    return (node_x, edge_flat.reshape(E, 2))


# speedup vs baseline: 28.2099x; 1.6581x over previous
import functools

import jax
import jax.numpy as jnp
from jax import lax
from jax.experimental import pallas as pl
from jax.experimental.pallas import tpu as pltpu
from jax.experimental.pallas import tpu_sc as plsc

N = 10000
E = 320000
D = 128
NC = 2
NS = 16
NW = NC * NS
IR = E // 128
RPW = IR // NW
TROW = NW * RPW
EPW = RPW * 128
DEG_PAD = NS * 640
NPAD = NS * 640

_MESH = plsc.VectorSubcoreMesh(core_axis_name="c", subcore_axis_name="s")
_F32 = jnp.float32
_I32 = jnp.int32


@functools.partial(
    pl.kernel,
    out_type=jax.ShapeDtypeStruct((NC * DEG_PAD,), _F32),
    mesh=_MESH,
    compiler_params=pltpu.CompilerParams(
        needs_layout_passes=False, use_tc_tiling_on_sc=False),
    scratch_types=[
        pltpu.VMEM((RPW, 128), _I32),
        pltpu.VMEM((1, 128), _I32),
        pltpu.VMEM((128,), _F32),
        pltpu.VMEM((640,), _F32),
        pltpu.VMEM_SHARED((DEG_PAD,), _F32),
    ],
)
def _sc_degree(dst2_hbm, out_hbm, idxb, tailb, onesv, zbuf, acc):
    c = lax.axis_index("c")
    s = lax.axis_index("s")
    w = c * NS + s

    zero16 = jnp.zeros((16,), _F32)
    one16 = jnp.ones((16,), _F32)
    for j in range(640 // 16):
        zbuf[pl.ds(j * 16, 16)] = zero16
    for j in range(128 // 16):
        onesv[pl.ds(j * 16, 16)] = one16
    pltpu.sync_copy(dst2_hbm.at[pl.ds(w * RPW, RPW), :], idxb)
    pltpu.sync_copy(zbuf, acc.at[pl.ds(s * 640, 640)])
    plsc.subcore_barrier()

    def body(j, _):
        pltpu.sync_copy(onesv, acc.at[idxb.at[j]], add=True)
        return ()

    lax.fori_loop(0, RPW, body, ())

    @pl.when(w < IR - TROW)
    def _():
        pltpu.sync_copy(dst2_hbm.at[pl.ds(TROW + w, 1), :], tailb)
        pltpu.sync_copy(onesv, acc.at[tailb.at[0]], add=True)

    plsc.subcore_barrier()
    pltpu.sync_copy(acc.at[pl.ds(s * 640, 640)],
                    out_hbm.at[pl.ds(c * DEG_PAD + s * 640, 640)])


@functools.partial(
    pl.kernel,
    out_type=jax.ShapeDtypeStruct((NC, NPAD, D), _F32),
    mesh=_MESH,
    compiler_params=pltpu.CompilerParams(
        needs_layout_passes=False, use_tc_tiling_on_sc=False),
    scratch_types=[
        pltpu.VMEM((RPW // 2, 128), _I32),
        pltpu.VMEM((RPW, 128), _I32),
        pltpu.VMEM((128, D), _F32),
        pltpu.VMEM((128, D), _F32),
        pltpu.VMEM_SHARED((NPAD, D), _F32),
        pltpu.SemaphoreType.DMA,
        pltpu.SemaphoreType.DMA,
    ],
)
def _sc_aggregate(table_hbm, src2_hbm, dst2_hbm, out_hbm,
                  srcb, dstb, rows0, rows1, acc, sg0, sg1):
    c = lax.axis_index("c")
    s = lax.axis_index("s")
    w = c * NS + s
    rowbufs = (rows0, rows1)
    sgs = (sg0, sg1)
    half = RPW // 2

    def gather_copy(jloc, b):
        return pltpu.make_async_copy(
            table_hbm.at[srcb.at[jloc]], rowbufs[b], sgs[b])

    zero16 = jnp.zeros((16,), _F32)

    def zbody(i, _):
        for j in range(D // 16):
            rows0[i, pl.ds(j * 16, 16)] = zero16
        return ()

    lax.fori_loop(0, 128, zbody, ())
    for k in range(5):
        pltpu.sync_copy(rows0,
                        acc.at[pl.ds(s * 640 + k * 128, 128), :])

    pltpu.sync_copy(src2_hbm.at[pl.ds(w * RPW, half), :], srcb)
    pltpu.sync_copy(dst2_hbm.at[pl.ds(w * RPW, RPW), :], dstb)

    plsc.subcore_barrier()

    for ph in range(2):
        if ph == 1:
            pltpu.sync_copy(
                src2_hbm.at[pl.ds(w * RPW + half, half), :], srcb)
        gather_copy(0, 0).start()

        def body(j, _, _ph=ph):
            b = lax.rem(j, 2)
            for bb in range(2):
                @pl.when(b == bb)
                def _():
                    gather_copy(j, bb).wait()

                    @pl.when(j + 1 < half)
                    def _():
                        gather_copy(j + 1, 1 - bb).start()
                    pltpu.sync_copy(
                        rowbufs[bb], acc.at[dstb.at[_ph * half + j]],
                        add=True)
            return ()

        lax.fori_loop(0, half, body, ())

    @pl.when(w < IR - TROW)
    def _():
        pltpu.sync_copy(src2_hbm.at[pl.ds(TROW + w, 1), :],
                        srcb.at[pl.ds(0, 1), :])
        pltpu.sync_copy(dst2_hbm.at[pl.ds(TROW + w, 1), :],
                        dstb.at[pl.ds(0, 1), :])
        pltpu.async_copy(table_hbm.at[srcb.at[0]], rows0, sg0).wait()
        pltpu.sync_copy(rows0, acc.at[dstb.at[0]], add=True)

    plsc.subcore_barrier()
    pltpu.sync_copy(acc.at[pl.ds(s * 640, 640), :],
                    out_hbm.at[c, pl.ds(s * 640, 640), :])


@functools.partial(
    pl.kernel,
    out_type=jax.ShapeDtypeStruct((2 * E,), _F32),
    mesh=_MESH,
    compiler_params=pltpu.CompilerParams(
        needs_layout_passes=False, use_tc_tiling_on_sc=False),
    scratch_types=[
        pltpu.VMEM((2 * N,), _F32),
        pltpu.VMEM((2 * N,), _F32),
        pltpu.VMEM((EPW,), _I32),
        pltpu.VMEM((EPW,), _I32),
        pltpu.VMEM((2 * EPW,), _F32),
        pltpu.VMEM((256,), _F32),
    ],
)
def _sc_edge_head(es_hbm, ed_hbm, src_hbm, dst_hbm, out_hbm,
                  esv, edv, srcv, dstv, outv, tailv):
    c = lax.axis_index("c")
    s = lax.axis_index("s")
    w = c * NS + s
    base = w * EPW
    pltpu.sync_copy(es_hbm, esv)
    pltpu.sync_copy(ed_hbm, edv)
    pltpu.sync_copy(src_hbm.at[pl.ds(base, EPW)], srcv)
    pltpu.sync_copy(dst_hbm.at[pl.ds(base, EPW)], dstv)

    ones16 = jnp.ones((16,), _I32)

    def do16(srcref, dstref, eoff, outref, ooff):
        s2 = srcref[pl.ds(eoff, 16)] * 2
        d2 = dstref[pl.ds(eoff, 16)] * 2
        o0 = plsc.load_gather(esv, [s2]) + plsc.load_gather(edv, [d2])
        o1 = (plsc.load_gather(esv, [s2 + ones16]) +
              plsc.load_gather(edv, [d2 + ones16]))
        outref[pl.ds(ooff, 16)] = o0
        outref[pl.ds(ooff + 128, 16)] = o1

    def body(jq, _):
        for jr in range(8):
            do16(srcv, dstv, pl.multiple_of(jq * 128 + jr * 16, 16),
                 outv, pl.multiple_of(jq * 256 + jr * 16, 16))
        return ()

    lax.fori_loop(0, RPW, body, ())
    pltpu.sync_copy(outv, out_hbm.at[pl.ds(2 * base, 2 * EPW)])

    @pl.when(w < IR - TROW)
    def _():
        tb = (TROW + w) * 128
        pltpu.sync_copy(src_hbm.at[pl.ds(tb, 128)], srcv.at[pl.ds(0, 128)])
        pltpu.sync_copy(dst_hbm.at[pl.ds(tb, 128)], dstv.at[pl.ds(0, 128)])
        for jr in range(8):
            do16(srcv, dstv, jr * 16, tailv, jr * 16)
        pltpu.sync_copy(tailv, out_hbm.at[pl.ds(2 * tb, 256)])


_RB = 1000
_GRID = N // _RB


def _row_spec(width):
    return pl.BlockSpec((_RB, width), lambda i: (i, 0))


def _full_spec(r, cdim):
    return pl.BlockSpec((r, cdim), lambda i: (0, 0))


def _dinv(p0, p1):
    return lax.rsqrt(p0 + p1 + 1.0)


def _tc_matmul_body(x_ref, w_ref, out_ref):
    out_ref[...] = jnp.dot(x_ref[...], w_ref[...],
                           preferred_element_type=_F32)


def _tc_scale1_body(xw_ref, p0_ref, p1_ref, out_ref):
    out_ref[...] = xw_ref[...] * _dinv(p0_ref[...], p1_ref[...])


def _tc_mid_body(a0_ref, a1_ref, s1_ref, p0_ref, p1_ref, w_ref, b1_ref,
                 out_ref):
    dinv = _dinv(p0_ref[...], p1_ref[...])
    pre = dinv * (a0_ref[...] + a1_ref[...] + s1_ref[...]) + b1_ref[...]
    h = jnp.maximum(pre, 0.0)
    out_ref[...] = dinv * jnp.dot(h, w_ref[...], preferred_element_type=_F32)


def _tc_head_body(a0_ref, a1_ref, s2_ref, p0_ref, p1_ref, b2_ref,
                  wn_ref, bn_ref, ws_ref, wd_ref, be_ref,
                  nx_ref, es_ref, ed_ref):
    dinv = _dinv(p0_ref[...], p1_ref[...])
    emb = dinv * (a0_ref[...] + a1_ref[...] + s2_ref[...]) + b2_ref[...]
    nx_ref[...] = jnp.dot(emb, wn_ref[...], preferred_element_type=_F32) + bn_ref[...]
    es_ref[...] = jnp.dot(emb, ws_ref[...], preferred_element_type=_F32) + be_ref[...]
    ed_ref[...] = jnp.dot(emb, wd_ref[...], preferred_element_type=_F32)


def kernel(x, edge_index, W1, b1, W2, b2, Wn, bn, We, be):
    src2 = edge_index[0].astype(_I32).reshape(IR, 128)
    dst2 = edge_index[1].astype(_I32).reshape(IR, 128)

    deg_parts = _sc_degree(dst2)
    p0 = deg_parts[:N].reshape(N, 1)
    p1 = deg_parts[DEG_PAD:DEG_PAD + N].reshape(N, 1)

    xw1 = pl.pallas_call(
        _tc_matmul_body,
        grid=(_GRID,),
        in_specs=[_row_spec(D), _full_spec(D, D)],
        out_specs=_row_spec(D),
        out_shape=jax.ShapeDtypeStruct((N, D), _F32),
    )(x, W1)

    scaled1 = pl.pallas_call(
        _tc_scale1_body,
        grid=(_GRID,),
        in_specs=[_row_spec(D), _row_spec(1), _row_spec(1)],
        out_specs=_row_spec(D),
        out_shape=jax.ShapeDtypeStruct((N, D), _F32),
    )(xw1, p0, p1)

    agg1 = _sc_aggregate(scaled1, src2, dst2)

    scaled2 = pl.pallas_call(
        _tc_mid_body,
        grid=(_GRID,),
        in_specs=[_row_spec(D), _row_spec(D), _row_spec(D), _row_spec(1),
                  _row_spec(1), _full_spec(D, D), _full_spec(1, D)],
        out_specs=_row_spec(D),
        out_shape=jax.ShapeDtypeStruct((N, D), _F32),
    )(agg1[0], agg1[1], scaled1, p0, p1, W2, b1.reshape(1, D))

    agg2 = _sc_aggregate(scaled2, src2, dst2)

    node_x, es, ed = pl.pallas_call(
        _tc_head_body,
        grid=(_GRID,),
        in_specs=[_row_spec(D), _row_spec(D), _row_spec(D), _row_spec(1),
                  _row_spec(1), _full_spec(1, D), _full_spec(D, 2),
                  _full_spec(1, 2), _full_spec(D, 2), _full_spec(D, 2),
                  _full_spec(1, 2)],
        out_specs=[_row_spec(2), _row_spec(2), _row_spec(2)],
        out_shape=[jax.ShapeDtypeStruct((N, 2), _F32),
                   jax.ShapeDtypeStruct((N, 2), _F32),
                   jax.ShapeDtypeStruct((N, 2), _F32)],
    )(agg2[0], agg2[1], scaled2, p0, p1, b2.reshape(1, D),
      Wn, bn.reshape(1, 2), We[:D], We[D:], be.reshape(1, 2))

    edge_flat = _sc_edge_head(
        es.reshape(-1), ed.reshape(-1),
        edge_index[0].astype(_I32), edge_index[1].astype(_I32))
    edge_x = edge_flat.reshape(IR, 2, 128).transpose(0, 2, 1).reshape(E, 2)
    return (node_x, edge_x)


# speedup vs baseline: 28.5201x; 1.0110x over previous
import functools

import jax
import jax.numpy as jnp
from jax import lax
from jax.experimental import pallas as pl
from jax.experimental.pallas import tpu as pltpu
from jax.experimental.pallas import tpu_sc as plsc

N = 10000
E = 320000
D = 128
NC = 2
NS = 16
NW = NC * NS
IR = E // 128
RPW = IR // NW
TROW = NW * RPW
EPW = RPW * 128
DEG_PAD = NS * 640
NPAD = NS * 640

_MESH = plsc.VectorSubcoreMesh(core_axis_name="c", subcore_axis_name="s")
_F32 = jnp.float32
_I32 = jnp.int32


@functools.partial(
    pl.kernel,
    out_type=jax.ShapeDtypeStruct((NC * DEG_PAD,), _F32),
    mesh=_MESH,
    compiler_params=pltpu.CompilerParams(
        needs_layout_passes=False, use_tc_tiling_on_sc=False),
    scratch_types=[
        pltpu.VMEM((RPW, 128), _I32),
        pltpu.VMEM((1, 128), _I32),
        pltpu.VMEM((128,), _F32),
        pltpu.VMEM((640,), _F32),
        pltpu.VMEM_SHARED((DEG_PAD,), _F32),
    ],
)
def _sc_degree(dst2_hbm, out_hbm, idxb, tailb, onesv, zbuf, acc):
    c = lax.axis_index("c")
    s = lax.axis_index("s")
    w = c * NS + s

    zero16 = jnp.zeros((16,), _F32)
    one16 = jnp.ones((16,), _F32)
    for j in range(640 // 16):
        zbuf[pl.ds(j * 16, 16)] = zero16
    for j in range(128 // 16):
        onesv[pl.ds(j * 16, 16)] = one16
    pltpu.sync_copy(dst2_hbm.at[pl.ds(w * RPW, RPW), :], idxb)
    pltpu.sync_copy(zbuf, acc.at[pl.ds(s * 640, 640)])
    plsc.subcore_barrier()

    def body(j, _):
        pltpu.sync_copy(onesv, acc.at[idxb.at[j]], add=True)
        return ()

    lax.fori_loop(0, RPW, body, ())

    @pl.when(w < IR - TROW)
    def _():
        pltpu.sync_copy(dst2_hbm.at[pl.ds(TROW + w, 1), :], tailb)
        pltpu.sync_copy(onesv, acc.at[tailb.at[0]], add=True)

    plsc.subcore_barrier()
    pltpu.sync_copy(acc.at[pl.ds(s * 640, 640)],
                    out_hbm.at[pl.ds(c * DEG_PAD + s * 640, 640)])


@functools.partial(
    pl.kernel,
    out_type=jax.ShapeDtypeStruct((NC, NPAD, D), _F32),
    mesh=_MESH,
    compiler_params=pltpu.CompilerParams(
        needs_layout_passes=False, use_tc_tiling_on_sc=False),
    scratch_types=[
        pltpu.VMEM((RPW // 2, 128), _I32),
        pltpu.VMEM((RPW, 128), _I32),
        pltpu.VMEM((128, D), _F32),
        pltpu.VMEM((128, D), _F32),
        pltpu.VMEM_SHARED((NPAD, D), _F32),
        pltpu.SemaphoreType.DMA,
        pltpu.SemaphoreType.DMA,
        pltpu.SemaphoreType.DMA,
    ],
)
def _sc_aggregate(table_hbm, src2_hbm, dst2_hbm, out_hbm,
                  srcb, dstb, rows0, rows1, acc, sg0, sg1, si):
    c = lax.axis_index("c")
    s = lax.axis_index("s")
    w = c * NS + s
    rowbufs = (rows0, rows1)
    sgs = (sg0, sg1)
    half = RPW // 2

    def gather_copy(jloc, b):
        return pltpu.make_async_copy(
            table_hbm.at[srcb.at[jloc]], rowbufs[b], sgs[b])

    src_cp = pltpu.make_async_copy(
        src2_hbm.at[pl.ds(w * RPW, half), :], srcb, si)
    src_cp.start()
    dst_cp = pltpu.make_async_copy(
        dst2_hbm.at[pl.ds(w * RPW, RPW), :], dstb, sg1)
    dst_cp.start()

    zero16 = jnp.zeros((16,), _F32)

    def zbody(i, _):
        for j in range(D // 16):
            rows0[i, pl.ds(j * 16, 16)] = zero16
        return ()

    lax.fori_loop(0, 128, zbody, ())
    for k in range(5):
        pltpu.sync_copy(rows0,
                        acc.at[pl.ds(s * 640 + k * 128, 128), :])

    src_cp.wait()
    dst_cp.wait()
    gather_copy(0, 0).start()

    plsc.subcore_barrier()

    for ph in range(2):
        if ph == 1:
            pltpu.sync_copy(
                src2_hbm.at[pl.ds(w * RPW + half, half), :], srcb)
            gather_copy(0, 0).start()

        def body(j, _, _ph=ph):
            b = lax.rem(j, 2)
            for bb in range(2):
                @pl.when(b == bb)
                def _():
                    gather_copy(j, bb).wait()

                    @pl.when(j + 1 < half)
                    def _():
                        gather_copy(j + 1, 1 - bb).start()
                    pltpu.sync_copy(
                        rowbufs[bb], acc.at[dstb.at[_ph * half + j]],
                        add=True)
            return ()

        lax.fori_loop(0, half, body, ())

    @pl.when(w < IR - TROW)
    def _():
        pltpu.sync_copy(src2_hbm.at[pl.ds(TROW + w, 1), :],
                        srcb.at[pl.ds(0, 1), :])
        pltpu.sync_copy(dst2_hbm.at[pl.ds(TROW + w, 1), :],
                        dstb.at[pl.ds(0, 1), :])
        pltpu.async_copy(table_hbm.at[srcb.at[0]], rows0, sg0).wait()
        pltpu.sync_copy(rows0, acc.at[dstb.at[0]], add=True)

    plsc.subcore_barrier()
    pltpu.sync_copy(acc.at[pl.ds(s * 640, 640), :],
                    out_hbm.at[c, pl.ds(s * 640, 640), :])


@functools.partial(
    pl.kernel,
    out_type=jax.ShapeDtypeStruct((2 * E,), _F32),
    mesh=_MESH,
    compiler_params=pltpu.CompilerParams(
        needs_layout_passes=False, use_tc_tiling_on_sc=False),
    scratch_types=[
        pltpu.VMEM((2 * N,), _F32),
        pltpu.VMEM((2 * N,), _F32),
        pltpu.VMEM((RPW, 128), _I32),
        pltpu.VMEM((RPW, 128), _I32),
        pltpu.VMEM((2 * EPW,), _F32),
        pltpu.VMEM((256,), _F32),
    ],
)
def _sc_edge_head(es_hbm, ed_hbm, src2_hbm, dst2_hbm, out_hbm,
                  esv, edv, srcv, dstv, outv, tailv):
    c = lax.axis_index("c")
    s = lax.axis_index("s")
    w = c * NS + s
    base = w * EPW
    pltpu.sync_copy(es_hbm, esv)
    pltpu.sync_copy(ed_hbm, edv)
    pltpu.sync_copy(src2_hbm.at[pl.ds(w * RPW, RPW), :], srcv)
    pltpu.sync_copy(dst2_hbm.at[pl.ds(w * RPW, RPW), :], dstv)

    ones16 = jnp.ones((16,), _I32)

    def do16(srcref, dstref, erow, eoff, outref, ooff):
        s2 = srcref[erow, pl.ds(eoff, 16)] * 2
        d2 = dstref[erow, pl.ds(eoff, 16)] * 2
        o0 = plsc.load_gather(esv, [s2]) + plsc.load_gather(edv, [d2])
        o1 = (plsc.load_gather(esv, [s2 + ones16]) +
              plsc.load_gather(edv, [d2 + ones16]))
        outref[pl.ds(ooff, 16)] = o0
        outref[pl.ds(ooff + 128, 16)] = o1

    def body(jq, _):
        for jr in range(8):
            do16(srcv, dstv, jq, jr * 16,
                 outv, pl.multiple_of(jq * 256 + jr * 16, 16))
        return ()

    lax.fori_loop(0, RPW, body, ())
    pltpu.sync_copy(outv, out_hbm.at[pl.ds(2 * base, 2 * EPW)])

    @pl.when(w < IR - TROW)
    def _():
        pltpu.sync_copy(src2_hbm.at[pl.ds(TROW + w, 1), :],
                        srcv.at[pl.ds(0, 1), :])
        pltpu.sync_copy(dst2_hbm.at[pl.ds(TROW + w, 1), :],
                        dstv.at[pl.ds(0, 1), :])
        for jr in range(8):
            do16(srcv, dstv, 0, jr * 16, tailv, jr * 16)
        pltpu.sync_copy(tailv, out_hbm.at[pl.ds(2 * (TROW + w) * 128, 256)])


_RB = 1000
_GRID = N // _RB


def _row_spec(width):
    return pl.BlockSpec((_RB, width), lambda i: (i, 0))


def _full_spec(r, cdim):
    return pl.BlockSpec((r, cdim), lambda i: (0, 0))


def _dinv(p0, p1):
    return lax.rsqrt(p0 + p1 + 1.0)


def _tc_matmul_body(x_ref, w_ref, out_ref):
    out_ref[...] = jnp.dot(x_ref[...], w_ref[...],
                           preferred_element_type=_F32)


def _tc_scale1_body(xw_ref, p0_ref, p1_ref, out_ref):
    out_ref[...] = xw_ref[...] * _dinv(p0_ref[...], p1_ref[...])


def _tc_mid_body(a0_ref, a1_ref, s1_ref, p0_ref, p1_ref, w_ref, b1_ref,
                 out_ref):
    dinv = _dinv(p0_ref[...], p1_ref[...])
    pre = dinv * (a0_ref[...] + a1_ref[...] + s1_ref[...]) + b1_ref[...]
    h = jnp.maximum(pre, 0.0)
    out_ref[...] = dinv * jnp.dot(h, w_ref[...], preferred_element_type=_F32)


def _tc_head_body(a0_ref, a1_ref, s2_ref, p0_ref, p1_ref, b2_ref,
                  wn_ref, bn_ref, ws_ref, wd_ref, be_ref,
                  nx_ref, es_ref, ed_ref):
    dinv = _dinv(p0_ref[...], p1_ref[...])
    emb = dinv * (a0_ref[...] + a1_ref[...] + s2_ref[...]) + b2_ref[...]
    nx_ref[...] = jnp.dot(emb, wn_ref[...], preferred_element_type=_F32) + bn_ref[...]
    es_ref[...] = jnp.dot(emb, ws_ref[...], preferred_element_type=_F32) + be_ref[...]
    ed_ref[...] = jnp.dot(emb, wd_ref[...], preferred_element_type=_F32)


def kernel(x, edge_index, W1, b1, W2, b2, Wn, bn, We, be):
    src2 = edge_index[0].astype(_I32).reshape(IR, 128)
    dst2 = edge_index[1].astype(_I32).reshape(IR, 128)

    deg_parts = _sc_degree(dst2)
    p0 = deg_parts[:N].reshape(N, 1)
    p1 = deg_parts[DEG_PAD:DEG_PAD + N].reshape(N, 1)

    xw1 = pl.pallas_call(
        _tc_matmul_body,
        grid=(_GRID,),
        in_specs=[_row_spec(D), _full_spec(D, D)],
        out_specs=_row_spec(D),
        out_shape=jax.ShapeDtypeStruct((N, D), _F32),
    )(x, W1)

    scaled1 = pl.pallas_call(
        _tc_scale1_body,
        grid=(_GRID,),
        in_specs=[_row_spec(D), _row_spec(1), _row_spec(1)],
        out_specs=_row_spec(D),
        out_shape=jax.ShapeDtypeStruct((N, D), _F32),
    )(xw1, p0, p1)

    agg1 = _sc_aggregate(scaled1, src2, dst2)

    scaled2 = pl.pallas_call(
        _tc_mid_body,
        grid=(_GRID,),
        in_specs=[_row_spec(D), _row_spec(D), _row_spec(D), _row_spec(1),
                  _row_spec(1), _full_spec(D, D), _full_spec(1, D)],
        out_specs=_row_spec(D),
        out_shape=jax.ShapeDtypeStruct((N, D), _F32),
    )(agg1[0], agg1[1], scaled1, p0, p1, W2, b1.reshape(1, D))

    agg2 = _sc_aggregate(scaled2, src2, dst2)

    node_x, es, ed = pl.pallas_call(
        _tc_head_body,
        grid=(_GRID,),
        in_specs=[_row_spec(D), _row_spec(D), _row_spec(D), _row_spec(1),
                  _row_spec(1), _full_spec(1, D), _full_spec(D, 2),
                  _full_spec(1, 2), _full_spec(D, 2), _full_spec(D, 2),
                  _full_spec(1, 2)],
        out_specs=[_row_spec(2), _row_spec(2), _row_spec(2)],
        out_shape=[jax.ShapeDtypeStruct((N, 2), _F32),
                   jax.ShapeDtypeStruct((N, 2), _F32),
                   jax.ShapeDtypeStruct((N, 2), _F32)],
    )(agg2[0], agg2[1], scaled2, p0, p1, b2.reshape(1, D),
      Wn, bn.reshape(1, 2), We[:D], We[D:], be.reshape(1, 2))

    edge_flat = _sc_edge_head(es.reshape(-1), ed.reshape(-1), src2, dst2)
    edge_x = edge_flat.reshape(IR, 2, 128).transpose(0, 2, 1).reshape(E, 2)
    return (node_x, edge_x)
